# baseline (speedup 1.0000x reference)
import functools

import jax
import jax.numpy as jnp
from jax import lax
from jax.experimental import pallas as pl
from jax.experimental.pallas import tpu as pltpu
from jax.experimental.pallas import tpu_sc as plsc

_HIDDEN = 2048
_EXPERTS = 64
_TOPK = 8
_TB = 2048
_N = 16384
_NC = 2
_NS = 16
_NW = _NC * _NS
_L = 16


def _matmul_body(x_ref, w_ref, logits_ref):
    logits_ref[...] = jnp.dot(x_ref[...], w_ref[...],
                              preferred_element_type=jnp.float32)


def _tc_logits(x, w):
    n = x.shape[0]
    return pl.pallas_call(
        _matmul_body,
        grid=(n // _TB,),
        in_specs=[
            pl.BlockSpec((_TB, _HIDDEN), lambda i: (i, 0)),
            pl.BlockSpec((_HIDDEN, _EXPERTS), lambda i: (0, 0)),
        ],
        out_specs=pl.BlockSpec((_TB, _EXPERTS), lambda i: (i, 0)),
        out_shape=jax.ShapeDtypeStruct((n, _EXPERTS), jnp.float32),
        compiler_params=pltpu.CompilerParams(
            dimension_semantics=("parallel",),
        ),
    )(x, w)


def _sc_topk_kernel(nt):
    mesh = plsc.VectorSubcoreMesh(core_axis_name="c", subcore_axis_name="s")
    tpw = nt // _NW

    @functools.partial(
        pl.kernel,
        mesh=mesh,
        out_type=[
            jax.ShapeDtypeStruct((nt * _TOPK,), jnp.int32),
            jax.ShapeDtypeStruct((nt * _TOPK,), jnp.float32),
        ],
        scratch_types=[
            pltpu.VMEM((tpw, _EXPERTS), jnp.float32),
            pltpu.VMEM((tpw * _TOPK,), jnp.int32),
            pltpu.VMEM((tpw * _TOPK,), jnp.float32),
        ],
        compiler_params=pltpu.CompilerParams(needs_layout_passes=False),
    )
    def sc_topk(logits_hbm, idx_hbm, wts_hbm, lg_v, idx_v, wts_v):
        wid = lax.axis_index("s") * _NC + lax.axis_index("c")
        base = wid * tpw
        pltpu.sync_copy(logits_hbm.at[pl.ds(base, tpw)], lg_v)

        iota_i = lax.iota(jnp.int32, _L)
        iota_f = iota_i.astype(jnp.float32)
        lane_lt8 = iota_i < _TOPK
        shift_idx = (iota_i + _TOPK) & (_L - 1)
        neg_inf = jnp.float32(-jnp.inf)

        def shift8(v):
            return lax.gather(
                v, shift_idx[:, None],
                dimension_numbers=lax.GatherDimensionNumbers(
                    offset_dims=(), collapsed_slice_dims=(0,),
                    start_index_map=(0,)),
                slice_sizes=(1,),
                mode=lax.GatherScatterMode.PROMISE_IN_BOUNDS)

        def merge(ak, av, bk, bv):
            ck = jnp.where(lane_lt8, ak, shift8(bk))
            cv = jnp.where(lane_lt8, av, shift8(bv))
            return plsc.sort_key_val(ck, cv, descending=True)

        def topk_one(t):
            ks, vs = [], []
            for j in range(_EXPERTS // _L):
                raw = lg_v[t, pl.ds(j * _L, _L)]
                adj = raw - (iota_f + jnp.float32(j * _L)) * jnp.float32(1e-9)
                gidx = iota_i + jnp.int32(j * _L)
                sk, sv = plsc.sort_key_val(adj, gidx, descending=True)
                ks.append(sk)
                vs.append(sv)
            k01, v01 = merge(ks[0], vs[0], ks[1], vs[1])
            k23, v23 = merge(ks[2], vs[2], ks[3], vs[3])
            tk, tv = merge(k01, v01, k23, v23)
            e = jnp.exp(tk - lax.broadcast_in_dim(
                lax.reduce_max(tk, (0,)), (_L,), ()))
            e = jnp.where(lane_lt8, e, jnp.float32(0.0))
            denom = lax.broadcast_in_dim(lax.reduce_sum(e, (0,)), (_L,), ())
            w = e / denom
            return tv, w

        def pair_body(p, carry):
            tv_a, w_a = topk_one(2 * p)
            tv_b, w_b = topk_one(2 * p + 1)
            idx_pair = jnp.where(lane_lt8, tv_a, shift8(tv_b))
            wts_pair = jnp.where(lane_lt8, w_a, shift8(w_b))
            idx_v[pl.ds(p * 2 * _TOPK, _L)] = idx_pair
            wts_v[pl.ds(p * 2 * _TOPK, _L)] = wts_pair
            return carry

        lax.fori_loop(0, tpw // 2, pair_body, 0, unroll=4)

        pltpu.sync_copy(idx_v, idx_hbm.at[pl.ds(base * _TOPK, tpw * _TOPK)])
        pltpu.sync_copy(wts_v, wts_hbm.at[pl.ds(base * _TOPK, tpw * _TOPK)])

    return sc_topk


_CHUNKS = 4
_NCHUNK = _N // _CHUNKS
_SC_TOPK = _sc_topk_kernel(_NCHUNK)


@functools.partial(jax.jit, static_argnames=())
def kernel(hidden_states, W_gate):
    b, s, h = hidden_states.shape
    n = b * s
    x = hidden_states.reshape(n, h)

    logits_c = []
    idx_c = []
    wts_c = []
    for c in range(_CHUNKS):
        lg = _tc_logits(
            lax.slice(x, (c * _NCHUNK, 0), ((c + 1) * _NCHUNK, h)), W_gate)
        i8, w8 = _SC_TOPK(lg)
        logits_c.append(lg)
        idx_c.append(i8)
        wts_c.append(w8)

    logits = jnp.concatenate(logits_c, axis=0)
    idx_flat = jnp.concatenate(idx_c, axis=0)
    wts_flat = jnp.concatenate(wts_c, axis=0)

    return (
        logits.reshape(b, s, _EXPERTS),
        idx_flat.reshape(b, s, _TOPK),
        wts_flat.reshape(b, s, _TOPK),
    )


# speedup vs baseline: 1.6863x; 1.6863x over previous
import functools

import jax
import jax.numpy as jnp
from jax import lax
from jax.experimental import pallas as pl
from jax.experimental.pallas import tpu as pltpu
from jax.experimental.pallas import tpu_sc as plsc

_HIDDEN = 2048
_EXPERTS = 64
_TOPK = 8
_TB = 2048
_N = 16384
_NC = 2
_NS = 16
_NW = _NC * _NS
_L = 16


def _matmul_body(x_ref, w_ref, logits_ref):
    logits_ref[...] = jnp.dot(x_ref[...], w_ref[...],
                              preferred_element_type=jnp.float32)


def _tc_logits(x, w):
    n = x.shape[0]
    return pl.pallas_call(
        _matmul_body,
        grid=(n // _TB,),
        in_specs=[
            pl.BlockSpec((_TB, _HIDDEN), lambda i: (i, 0)),
            pl.BlockSpec((_HIDDEN, _EXPERTS), lambda i: (0, 0)),
        ],
        out_specs=pl.BlockSpec((_TB, _EXPERTS), lambda i: (i, 0)),
        out_shape=jax.ShapeDtypeStruct((n, _EXPERTS), jnp.float32),
        compiler_params=pltpu.CompilerParams(
            dimension_semantics=("parallel",),
        ),
    )(x, w)


def _sc_topk_kernel(nt):
    mesh = plsc.VectorSubcoreMesh(core_axis_name="c", subcore_axis_name="s")
    tpw = nt // _NW

    @functools.partial(
        pl.kernel,
        mesh=mesh,
        out_type=[
            jax.ShapeDtypeStruct((nt * _TOPK,), jnp.int32),
            jax.ShapeDtypeStruct((nt * _TOPK,), jnp.float32),
        ],
        scratch_types=[
            pltpu.VMEM((tpw, _EXPERTS), jnp.float32),
            pltpu.VMEM((tpw * _TOPK,), jnp.int32),
            pltpu.VMEM((tpw * _TOPK,), jnp.float32),
        ],
        compiler_params=pltpu.CompilerParams(needs_layout_passes=False),
    )
    def sc_topk(logits_hbm, idx_hbm, wts_hbm, lg_v, idx_v, wts_v):
        wid = lax.axis_index("s") * _NC + lax.axis_index("c")
        base = wid * tpw
        pltpu.sync_copy(logits_hbm.at[pl.ds(base, tpw)], lg_v)

        iota_i = lax.iota(jnp.int32, _L)
        iota_f = iota_i.astype(jnp.float32)
        lane_lt8 = iota_i < _TOPK
        shift_idx = (iota_i + _TOPK) & (_L - 1)
        zero_idx = iota_i & jnp.int32(0)
        biases = [(iota_f + jnp.float32(j * _L)) * jnp.float32(1e-9)
                  for j in range(_EXPERTS // _L)]
        gidxs = [iota_i + jnp.int32(j * _L) for j in range(_EXPERTS // _L)]

        def lane_perm(v, idx):
            return lax.gather(
                v, idx[:, None],
                dimension_numbers=lax.GatherDimensionNumbers(
                    offset_dims=(), collapsed_slice_dims=(0,),
                    start_index_map=(0,)),
                slice_sizes=(1,),
                mode=lax.GatherScatterMode.PROMISE_IN_BOUNDS)

        def shift8(v):
            return lane_perm(v, shift_idx)

        def merge(ak, av, bk, bv):
            ck = jnp.where(lane_lt8, ak, shift8(bk))
            cv = jnp.where(lane_lt8, av, shift8(bv))
            return plsc.sort_key_val(ck, cv, descending=True)

        def topk_one(t):
            ks, vs = [], []
            for j in range(_EXPERTS // _L):
                raw = lg_v[t, pl.ds(j * _L, _L)]
                adj = raw - biases[j]
                sk, sv = plsc.sort_key_val(adj, gidxs[j], descending=True)
                ks.append(sk)
                vs.append(sv)
            k01, v01 = merge(ks[0], vs[0], ks[1], vs[1])
            k23, v23 = merge(ks[2], vs[2], ks[3], vs[3])
            tk, tv = merge(k01, v01, k23, v23)
            e = jnp.exp(tk - lane_perm(tk, zero_idx))
            e = jnp.where(lane_lt8, e, jnp.float32(0.0))
            denom = lax.broadcast_in_dim(lax.reduce_sum(e, (0,)), (_L,), ())
            w = e / denom
            return tv, w

        def pair_body(p, carry):
            tv_a, w_a = topk_one(2 * p)
            tv_b, w_b = topk_one(2 * p + 1)
            idx_pair = jnp.where(lane_lt8, tv_a, shift8(tv_b))
            wts_pair = jnp.where(lane_lt8, w_a, shift8(w_b))
            idx_v[pl.ds(p * 2 * _TOPK, _L)] = idx_pair
            wts_v[pl.ds(p * 2 * _TOPK, _L)] = wts_pair
            return carry

        lax.fori_loop(0, tpw // 2, pair_body, 0, unroll=8)

        pltpu.sync_copy(idx_v, idx_hbm.at[pl.ds(base * _TOPK, tpw * _TOPK)])
        pltpu.sync_copy(wts_v, wts_hbm.at[pl.ds(base * _TOPK, tpw * _TOPK)])

    return sc_topk


_SC_TOPK = _sc_topk_kernel(_N)


@functools.partial(jax.jit, static_argnames=())
def kernel(hidden_states, W_gate):
    b, s, h = hidden_states.shape
    n = b * s
    x = hidden_states.reshape(n, h)

    logits = _tc_logits(x, W_gate)
    idx_flat, wts_flat = _SC_TOPK(logits)

    return (
        logits.reshape(b, s, _EXPERTS),
        idx_flat.reshape(b, s, _TOPK),
        wts_flat.reshape(b, s, _TOPK),
    )


# speedup vs baseline: 1.9724x; 1.1696x over previous
import functools

import jax
import jax.numpy as jnp
from jax import lax
from jax.experimental import pallas as pl
from jax.experimental.pallas import tpu as pltpu
from jax.experimental.pallas import tpu_sc as plsc

_HIDDEN = 2048
_EXPERTS = 64
_TOPK = 8
_TB = 2048
_N = 16384
_NC = 2
_NS = 16
_NW = _NC * _NS
_L = 16


def _matmul_body(x_ref, w_ref, logits_ref):
    logits_ref[...] = jnp.dot(x_ref[...], w_ref[...],
                              preferred_element_type=jnp.float32)


def _tc_logits(x, w):
    n = x.shape[0]
    return pl.pallas_call(
        _matmul_body,
        grid=(n // _TB,),
        in_specs=[
            pl.BlockSpec((_TB, _HIDDEN), lambda i: (i, 0)),
            pl.BlockSpec((_HIDDEN, _EXPERTS), lambda i: (0, 0)),
        ],
        out_specs=pl.BlockSpec((_TB, _EXPERTS), lambda i: (i, 0)),
        out_shape=jax.ShapeDtypeStruct((n, _EXPERTS), jnp.float32),
        compiler_params=pltpu.CompilerParams(
            dimension_semantics=("parallel",),
        ),
    )(x, w)


def _sc_topk_kernel(nt):
    mesh = plsc.VectorSubcoreMesh(core_axis_name="c", subcore_axis_name="s")
    tpw = nt // _NW

    @functools.partial(
        pl.kernel,
        mesh=mesh,
        out_type=[
            jax.ShapeDtypeStruct((nt * _TOPK,), jnp.int32),
            jax.ShapeDtypeStruct((nt * _TOPK,), jnp.float32),
        ],
        scratch_types=[
            pltpu.VMEM((tpw, _EXPERTS), jnp.float32),
            pltpu.VMEM((tpw * _TOPK,), jnp.int32),
            pltpu.VMEM((tpw * _TOPK,), jnp.float32),
        ],
        compiler_params=pltpu.CompilerParams(needs_layout_passes=False),
    )
    def sc_topk(logits_hbm, idx_hbm, wts_hbm, lg_v, idx_v, wts_v):
        wid = lax.axis_index("s") * _NC + lax.axis_index("c")
        base = wid * tpw
        pltpu.sync_copy(logits_hbm.at[pl.ds(base, tpw)], lg_v)

        iota_i = lax.iota(jnp.int32, _L)
        iota_f = iota_i.astype(jnp.float32)
        lane_lt8 = iota_i < _TOPK
        shift_idx = (iota_i + _TOPK) & (_L - 1)
        zero_idx = iota_i & jnp.int32(0)
        biases = [(iota_f + jnp.float32(j * _L)) * jnp.float32(1e-9)
                  for j in range(_EXPERTS // _L)]
        gidxs = [iota_i + jnp.int32(j * _L) for j in range(_EXPERTS // _L)]

        def lane_perm(v, idx):
            return lax.gather(
                v, idx[:, None],
                dimension_numbers=lax.GatherDimensionNumbers(
                    offset_dims=(), collapsed_slice_dims=(0,),
                    start_index_map=(0,)),
                slice_sizes=(1,),
                mode=lax.GatherScatterMode.PROMISE_IN_BOUNDS)

        def shift8(v):
            return lane_perm(v, shift_idx)

        def merge(ak, av, bk, bv):
            ck = jnp.where(lane_lt8, ak, shift8(bk))
            cv = jnp.where(lane_lt8, av, shift8(bv))
            return plsc.sort_key_val(ck, cv, descending=True)

        def topk_one(t):
            ks, vs = [], []
            for j in range(_EXPERTS // _L):
                raw = lg_v[t, pl.ds(j * _L, _L)]
                adj = raw - biases[j]
                sk, sv = plsc.sort_key_val(adj, gidxs[j], descending=True)
                ks.append(sk)
                vs.append(sv)
            k01, v01 = merge(ks[0], vs[0], ks[1], vs[1])
            k23, v23 = merge(ks[2], vs[2], ks[3], vs[3])
            tk, tv = merge(k01, v01, k23, v23)
            e = jnp.exp(tk - lane_perm(tk, zero_idx))
            e = jnp.where(lane_lt8, e, jnp.float32(0.0))
            denom = lax.broadcast_in_dim(lax.reduce_sum(e, (0,)), (_L,), ())
            w = e / denom
            return tv, w

        @plsc.parallel_loop(0, tpw // 2, unroll=8)
        def pair_body(p):
            tv_a, w_a = topk_one(2 * p)
            tv_b, w_b = topk_one(2 * p + 1)
            idx_pair = jnp.where(lane_lt8, tv_a, shift8(tv_b))
            wts_pair = jnp.where(lane_lt8, w_a, shift8(w_b))
            idx_v[pl.ds(p * 2 * _TOPK, _L)] = idx_pair
            wts_v[pl.ds(p * 2 * _TOPK, _L)] = wts_pair

        pltpu.sync_copy(idx_v, idx_hbm.at[pl.ds(base * _TOPK, tpw * _TOPK)])
        pltpu.sync_copy(wts_v, wts_hbm.at[pl.ds(base * _TOPK, tpw * _TOPK)])

    return sc_topk


_SC_TOPK = _sc_topk_kernel(_N)


@functools.partial(jax.jit, static_argnames=())
def kernel(hidden_states, W_gate):
    b, s, h = hidden_states.shape
    n = b * s
    x = hidden_states.reshape(n, h)

    logits = _tc_logits(x, W_gate)
    idx_flat, wts_flat = _SC_TOPK(logits)

    return (
        logits.reshape(b, s, _EXPERTS),
        idx_flat.reshape(b, s, _TOPK),
        wts_flat.reshape(b, s, _TOPK),
    )
